# baseline (speedup 1.0000x reference)
import functools

import jax
import jax.numpy as jnp
from jax import lax
from jax.experimental import pallas as pl
from jax.experimental.pallas import tpu as pltpu
from jax.experimental.pallas import tpu_sc as plsc

B, L, D = 4096, 200, 64
FLAT = L * D
LANES = 16
VECS = FLAT // LANES
NC = 2
NS = 16
NW = NC * NS
RPW = B // NW
CH = 4
NCH = RPW // CH


def _body(x_hbm, pos_hbm, out_hbm, pos_v, buf_v):
    wid = lax.axis_index("s") * NC + lax.axis_index("c")
    base = wid * RPW
    pltpu.sync_copy(pos_hbm, pos_v)

    def chunk_body(i, carry):
        row0 = base + i * CH
        pltpu.sync_copy(x_hbm.at[pl.ds(row0, CH)], buf_v)

        def vec_body(j, c2):
            p = pos_v[pl.ds(j * LANES, LANES)]
            for r in range(CH):
                plsc.addupdate(buf_v.at[r, pl.ds(j * LANES, LANES)], p)
            return c2

        lax.fori_loop(0, VECS, vec_body, 0)
        pltpu.sync_copy(buf_v, out_hbm.at[pl.ds(row0, CH)])
        return carry

    lax.fori_loop(0, NCH, chunk_body, 0)


def kernel(x, pos_table):
    xf = x.reshape(B, FLAT)
    pf = pos_table.reshape(FLAT)
    mesh = plsc.VectorSubcoreMesh(core_axis_name="c", subcore_axis_name="s")
    run = functools.partial(
        pl.kernel,
        mesh=mesh,
        out_type=jax.ShapeDtypeStruct((B, FLAT), jnp.float32),
        scratch_types=[
            pltpu.VMEM((FLAT,), jnp.float32),
            pltpu.VMEM((CH, FLAT), jnp.float32),
        ],
    )(_body)
    out = run(xf, pf)
    return out.reshape(B, L, D)
